# Initial kernel scaffold; baseline (speedup 1.0000x reference)
import jax
import jax.numpy as jnp
from jax.experimental import pallas as pl


def kernel(ch1, ch2, theta, d):
    raise NotImplementedError("write your pallas kernel here")


# speedup vs baseline: 3.2505x; 3.2505x over previous
import functools

import jax
import jax.numpy as jnp
from jax import lax
from jax.experimental import pallas as pl
from jax.experimental.pallas import tpu as pltpu
from jax.experimental.pallas import tpu_sc as plsc

N = 8192
K = 8
L = 16
NW = 32
QPW = N // NW
NKV = N // L
INV_SIG4 = 1.0 / 4.5


def _sc_body(ch1x_h, ch1y_h, ch2x_h, ch2y_h, prm_h, out_h,
             kx, ky, qx, qy, prm, sv):
    wid = lax.axis_index("s") * 2 + lax.axis_index("c")
    base = wid * QPW

    pltpu.sync_copy(ch2x_h, kx)
    pltpu.sync_copy(ch2y_h, ky)
    pltpu.sync_copy(ch1x_h.at[pl.ds(base, QPW)], qx)
    pltpu.sync_copy(ch1y_h.at[pl.ds(base, QPW)], qy)
    pltpu.sync_copy(prm_h, prm)

    cv = prm[0, :]
    sn = prm[1, :]
    dxv = prm[2, :]
    dyv = prm[3, :]

    def xform(i, _):
        x = kx[pl.ds(i * L, L)] - dxv
        y = ky[pl.ds(i * L, L)] - dyv
        kx[pl.ds(i * L, L)] = x * cv + y * sn
        ky[pl.ds(i * L, L)] = y * cv - x * sn
        return 0

    lax.fori_loop(0, NKV, xform, 0, unroll=2)

    inf = jnp.float32(jnp.inf)
    lane = lax.iota(jnp.int32, L)
    perms = [jnp.bitwise_xor(lane, sh) for sh in (1, 2, 4, 8)]

    def xlmin(v):
        for p in perms:
            v = jnp.minimum(v, v.at[p].get(mode="promise_in_bounds"))
        return v

    def xladd(v):
        for p in perms:
            v = v + v.at[p].get(mode="promise_in_bounds")
        return v

    def chunk(c, _):
        res = jnp.zeros((L,), jnp.float32)
        qxc = qx[pl.ds(c * L, L)]
        qyc = qy[pl.ds(c * L, L)]
        for j in range(L):
            jv = jnp.full((L,), j, jnp.int32)
            qxs = qxc.at[jv].get(mode="promise_in_bounds")
            qys = qyc.at[jv].get(mode="promise_in_bounds")

            def scan_keys(i, s):
                s0, s1, s2, s3, s4, s5, s6, s7 = s
                dx = kx[pl.ds(i * L, L)] - qxs
                dy = ky[pl.ds(i * L, L)] - qys
                v = dx * dx + dy * dy
                n0 = jnp.minimum(s0, v); v = jnp.maximum(s0, v)
                n1 = jnp.minimum(s1, v); v = jnp.maximum(s1, v)
                n2 = jnp.minimum(s2, v); v = jnp.maximum(s2, v)
                n3 = jnp.minimum(s3, v); v = jnp.maximum(s3, v)
                n4 = jnp.minimum(s4, v); v = jnp.maximum(s4, v)
                n5 = jnp.minimum(s5, v); v = jnp.maximum(s5, v)
                n6 = jnp.minimum(s6, v); v = jnp.maximum(s6, v)
                n7 = jnp.minimum(s7, v)
                return (n0, n1, n2, n3, n4, n5, n6, n7)

            init = (jnp.full((L,), inf),) * 8
            s = lax.fori_loop(0, NKV, scan_keys, init, unroll=2)

            def extract(t, carry):
                s0, s1, s2, s3, s4, s5, s6, s7, accv, remv = carry
                me = jnp.minimum(jnp.minimum(jnp.minimum(s0, s1),
                                             jnp.minimum(s2, s3)),
                                 jnp.minimum(jnp.minimum(s4, s5),
                                             jnp.minimum(s6, s7)))
                ms = xlmin(me)
                cnt = jnp.zeros((L,), jnp.float32)
                outs = []
                for r in (s0, s1, s2, s3, s4, s5, s6, s7):
                    msk = r == ms
                    cnt = cnt + jnp.where(msk, 1.0, 0.0)
                    outs.append(jnp.where(msk, inf, r))
                cs = xladd(cnt)
                take = jnp.minimum(cs, remv)
                accv = accv + take * jnp.exp(ms * (-INV_SIG4))
                remv = remv - take
                return tuple(outs) + (accv, remv)

            carry0 = s + (jnp.zeros((L,), jnp.float32),
                          jnp.full((L,), float(K), jnp.float32))
            fin = lax.fori_loop(0, K, extract, carry0)
            accv = fin[8]
            res = jnp.where(lane == j, accv, res)
        sv[pl.ds(c * L, L)] = res
        return 0

    lax.fori_loop(0, QPW // L, chunk, 0)
    pltpu.sync_copy(sv, out_h.at[pl.ds(base, QPW)])


def _sc_knn(ch1x, ch1y, ch2x, ch2y, prm):
    mesh = plsc.VectorSubcoreMesh(core_axis_name="c", subcore_axis_name="s")
    f = functools.partial(
        pl.kernel,
        mesh=mesh,
        out_type=jax.ShapeDtypeStruct((N,), jnp.float32),
        scratch_types=[
            pltpu.VMEM((N,), jnp.float32),
            pltpu.VMEM((N,), jnp.float32),
            pltpu.VMEM((QPW,), jnp.float32),
            pltpu.VMEM((QPW,), jnp.float32),
            pltpu.VMEM((4, L), jnp.float32),
            pltpu.VMEM((QPW,), jnp.float32),
        ],
    )(_sc_body)
    return f(ch1x, ch1y, ch2x, ch2y, prm)


def _post_body(s_ref, o_ref):
    e = s_ref[...] * (1.0 / N)
    mask = e != 0.0
    safe = jnp.where(mask, e, 1.0)
    o_ref[0, 0] = -jnp.sum(jnp.where(mask, jnp.log(safe), 0.0))


def _post(svals):
    out = pl.pallas_call(
        _post_body,
        out_shape=jax.ShapeDtypeStruct((1, 1), jnp.float32),
        out_specs=pl.BlockSpec(memory_space=pltpu.SMEM),
    )(svals.reshape(8, N // 8))
    return out[0, 0]


def kernel(ch1, ch2, theta, d):
    ch1 = jnp.asarray(ch1, jnp.float32)
    ch2 = jnp.asarray(ch2, jnp.float32)
    th = theta[0]
    c = jnp.cos(th)
    s = jnp.sin(th)
    prm = jnp.stack([
        jnp.broadcast_to(c, (L,)),
        jnp.broadcast_to(s, (L,)),
        jnp.broadcast_to(d[0], (L,)),
        jnp.broadcast_to(d[1], (L,)),
    ])
    ch1x = ch1[:, 0]
    ch1y = ch1[:, 1]
    ch2x = ch2[:, 0]
    ch2y = ch2[:, 1]
    svals = _sc_knn(ch1x, ch1y, ch2x, ch2y, prm)
    return _post(svals)


# speedup vs baseline: 5.3026x; 1.6313x over previous
import functools

import jax
import jax.numpy as jnp
from jax import lax
from jax.experimental import pallas as pl
from jax.experimental.pallas import tpu as pltpu
from jax.experimental.pallas import tpu_sc as plsc

N = 8192
K = 8
L = 16
NW = 32
QSC = 2048
QTC = N - QSC
QPW = QSC // NW
NKV = N // L
TCB = 1024
INV_SIG4 = 1.0 / 4.5


def _sc_body(ch1x_h, ch1y_h, ch2x_h, ch2y_h, prm_h, out_h,
             kx, ky, qx, qy, prm, sv):
    wid = lax.axis_index("s") * 2 + lax.axis_index("c")
    base = wid * QPW

    pltpu.sync_copy(ch2x_h, kx)
    pltpu.sync_copy(ch2y_h, ky)
    pltpu.sync_copy(ch1x_h.at[pl.ds(base, QPW)], qx)
    pltpu.sync_copy(ch1y_h.at[pl.ds(base, QPW)], qy)
    pltpu.sync_copy(prm_h, prm)

    cv = prm[0, :]
    sn = prm[1, :]
    dxv = prm[2, :]
    dyv = prm[3, :]

    def xform(i, _):
        x = kx[pl.ds(i * L, L)] - dxv
        y = ky[pl.ds(i * L, L)] - dyv
        kx[pl.ds(i * L, L)] = x * cv + y * sn
        ky[pl.ds(i * L, L)] = y * cv - x * sn
        return 0

    lax.fori_loop(0, NKV, xform, 0, unroll=2)

    inf = jnp.float32(jnp.inf)
    lane = lax.iota(jnp.int32, L)
    perms = [jnp.bitwise_xor(lane, sh) for sh in (1, 2, 4, 8)]

    def xlmin(v):
        for p in perms:
            v = jnp.minimum(v, v.at[p].get(mode="promise_in_bounds"))
        return v

    def xladd(v):
        for p in perms:
            v = v + v.at[p].get(mode="promise_in_bounds")
        return v

    NQI = 2

    def insert8(s, v):
        s0, s1, s2, s3, s4, s5, s6, s7 = s
        n0 = jnp.minimum(s0, v); v = jnp.maximum(s0, v)
        n1 = jnp.minimum(s1, v); v = jnp.maximum(s1, v)
        n2 = jnp.minimum(s2, v); v = jnp.maximum(s2, v)
        n3 = jnp.minimum(s3, v); v = jnp.maximum(s3, v)
        n4 = jnp.minimum(s4, v); v = jnp.maximum(s4, v)
        n5 = jnp.minimum(s5, v); v = jnp.maximum(s5, v)
        n6 = jnp.minimum(s6, v); v = jnp.maximum(s6, v)
        n7 = jnp.minimum(s7, v)
        return (n0, n1, n2, n3, n4, n5, n6, n7)

    def merge_top8(s):
        def extract(t, carry):
            s_ = carry[:8]
            accv, remv = carry[8], carry[9]
            s0, s1, s2, s3, s4, s5, s6, s7 = s_
            me = jnp.minimum(jnp.minimum(jnp.minimum(s0, s1),
                                         jnp.minimum(s2, s3)),
                             jnp.minimum(jnp.minimum(s4, s5),
                                         jnp.minimum(s6, s7)))
            ms = xlmin(me)
            cnt = jnp.zeros((L,), jnp.float32)
            outs = []
            for r in s_:
                msk = r == ms
                cnt = cnt + jnp.where(msk, 1.0, 0.0)
                outs.append(jnp.where(msk, inf, r))
            cs = xladd(cnt)
            take = jnp.minimum(cs, remv)
            accv = accv + take * jnp.exp(ms * (-INV_SIG4))
            remv = remv - take
            return tuple(outs) + (accv, remv)

        carry0 = s + (jnp.zeros((L,), jnp.float32),
                      jnp.full((L,), float(K), jnp.float32))
        fin = lax.fori_loop(0, K, extract, carry0)
        return fin[8]

    def chunk(c, _):
        res = jnp.zeros((L,), jnp.float32)
        qxc = qx[pl.ds(c * L, L)]
        qyc = qy[pl.ds(c * L, L)]
        for j0 in range(0, L, NQI):
            qxs = []
            qys = []
            for j in range(j0, j0 + NQI):
                jv = jnp.full((L,), j, jnp.int32)
                qxs.append(qxc.at[jv].get(mode="promise_in_bounds"))
                qys.append(qyc.at[jv].get(mode="promise_in_bounds"))

            def scan_keys(i, s):
                kxv = kx[pl.ds(i * L, L)]
                kyv = ky[pl.ds(i * L, L)]
                out = ()
                for q in range(NQI):
                    dx = kxv - qxs[q]
                    dy = kyv - qys[q]
                    v = dx * dx + dy * dy
                    out = out + insert8(s[8 * q:8 * q + 8], v)
                return out

            init = (jnp.full((L,), inf),) * (8 * NQI)
            s = lax.fori_loop(0, NKV, scan_keys, init, unroll=2)

            for q in range(NQI):
                accv = merge_top8(s[8 * q:8 * q + 8])
                res = jnp.where(lane == (j0 + q), accv, res)
        sv[pl.ds(c * L, L)] = res
        return 0

    lax.fori_loop(0, QPW // L, chunk, 0)
    pltpu.sync_copy(sv, out_h.at[pl.ds(base, QPW)])


def _sc_knn(ch1x, ch1y, ch2x, ch2y, prm):
    mesh = plsc.VectorSubcoreMesh(core_axis_name="c", subcore_axis_name="s")
    f = functools.partial(
        pl.kernel,
        mesh=mesh,
        out_type=jax.ShapeDtypeStruct((QSC,), jnp.float32),
        scratch_types=[
            pltpu.VMEM((N,), jnp.float32),
            pltpu.VMEM((N,), jnp.float32),
            pltpu.VMEM((QPW,), jnp.float32),
            pltpu.VMEM((QPW,), jnp.float32),
            pltpu.VMEM((4, L), jnp.float32),
            pltpu.VMEM((QPW,), jnp.float32),
        ],
    )(_sc_body)
    return f(ch1x, ch1y, ch2x, ch2y, prm)


def _tc_body(prm_ref, qx_ref, qy_ref, kx_ref, ky_ref, o_ref):
    c = prm_ref[0, 0]
    s = prm_ref[0, 1]
    dx0 = prm_ref[0, 2]
    dy0 = prm_ref[0, 3]

    qxv = qx_ref[...]
    qyv = qy_ref[...]
    inf = jnp.float32(jnp.inf)

    def scan(j, sregs):
        xs = kx_ref[0, j] - dx0
        ys = ky_ref[0, j] - dy0
        bx = xs * c + ys * s
        by = ys * c - xs * s
        ddx = qxv - bx
        ddy = qyv - by
        v = ddx * ddx + ddy * ddy
        s0, s1, s2, s3, s4, s5, s6, s7 = sregs
        n0 = jnp.minimum(s0, v); v = jnp.maximum(s0, v)
        n1 = jnp.minimum(s1, v); v = jnp.maximum(s1, v)
        n2 = jnp.minimum(s2, v); v = jnp.maximum(s2, v)
        n3 = jnp.minimum(s3, v); v = jnp.maximum(s3, v)
        n4 = jnp.minimum(s4, v); v = jnp.maximum(s4, v)
        n5 = jnp.minimum(s5, v); v = jnp.maximum(s5, v)
        n6 = jnp.minimum(s6, v); v = jnp.maximum(s6, v)
        n7 = jnp.minimum(s7, v)
        return (n0, n1, n2, n3, n4, n5, n6, n7)

    init = (jnp.full((8, 128), inf),) * 8
    sregs = lax.fori_loop(0, N, scan, init, unroll=4)
    acc = jnp.zeros((8, 128), jnp.float32)
    for t in sregs:
        acc = acc + jnp.exp(t * (-INV_SIG4))
    o_ref[...] = acc


def _tc_knn(qx, qy, kx2d, ky2d, prm_row):
    nb = QTC // TCB
    out = pl.pallas_call(
        _tc_body,
        grid=(nb,),
        in_specs=[
            pl.BlockSpec(memory_space=pltpu.SMEM),
            pl.BlockSpec((8, 128), lambda i: (0, i)),
            pl.BlockSpec((8, 128), lambda i: (0, i)),
            pl.BlockSpec(memory_space=pltpu.SMEM),
            pl.BlockSpec(memory_space=pltpu.SMEM),
        ],
        out_specs=pl.BlockSpec((8, 128), lambda i: (0, i)),
        out_shape=jax.ShapeDtypeStruct((8, QTC // 8), jnp.float32),
    )(prm_row, qx.reshape(8, QTC // 8), qy.reshape(8, QTC // 8),
      kx2d, ky2d)
    return out.reshape(-1)


def _post_body(s_ref, o_ref):
    e = s_ref[...] * (1.0 / N)
    mask = e != 0.0
    safe = jnp.where(mask, e, 1.0)
    o_ref[0, 0] = -jnp.sum(jnp.where(mask, jnp.log(safe), 0.0))


def _post(svals):
    out = pl.pallas_call(
        _post_body,
        out_shape=jax.ShapeDtypeStruct((1, 1), jnp.float32),
        out_specs=pl.BlockSpec(memory_space=pltpu.SMEM),
    )(svals.reshape(8, N // 8))
    return out[0, 0]


def kernel(ch1, ch2, theta, d):
    ch1 = jnp.asarray(ch1, jnp.float32)
    ch2 = jnp.asarray(ch2, jnp.float32)
    th = theta[0]
    c = jnp.cos(th)
    s = jnp.sin(th)
    prm = jnp.stack([
        jnp.broadcast_to(c, (L,)),
        jnp.broadcast_to(s, (L,)),
        jnp.broadcast_to(d[0], (L,)),
        jnp.broadcast_to(d[1], (L,)),
    ])
    prm_row = jnp.stack([c, s, d[0], d[1]]).reshape(1, 4)
    ch1x = ch1[:, 0]
    ch1y = ch1[:, 1]
    ch2x = ch2[:, 0]
    ch2y = ch2[:, 1]
    s_sc = _sc_knn(ch1x[:QSC], ch1y[:QSC], ch2x, ch2y, prm)
    s_tc = _tc_knn(ch1x[QSC:], ch1y[QSC:], ch2x.reshape(1, N),
                   ch2y.reshape(1, N), prm_row)
    svals = jnp.concatenate([s_sc, s_tc])
    return _post(svals)


# speedup vs baseline: 7.1346x; 1.3455x over previous
import functools

import jax
import jax.numpy as jnp
from jax import lax
from jax.experimental import pallas as pl
from jax.experimental.pallas import tpu as pltpu
from jax.experimental.pallas import tpu_sc as plsc

N = 8192
K = 8
L = 16
NW = 32
QSC = 2048
QTC = N - QSC
QPW = QSC // NW
NKV = N // L
TCB = 1024
INV_SIG4 = 1.0 / 4.5


def _sc_body(ch1x_h, ch1y_h, ch2x_h, ch2y_h, prm_h, out_h,
             kx, ky, qx, qy, prm, sv):
    wid = lax.axis_index("s") * 2 + lax.axis_index("c")
    base = wid * QPW

    pltpu.sync_copy(ch2x_h, kx)
    pltpu.sync_copy(ch2y_h, ky)
    pltpu.sync_copy(ch1x_h.at[pl.ds(base, QPW)], qx)
    pltpu.sync_copy(ch1y_h.at[pl.ds(base, QPW)], qy)
    pltpu.sync_copy(prm_h, prm)

    cv = prm[0, :]
    sn = prm[1, :]
    dxv = prm[2, :]
    dyv = prm[3, :]

    def xform(i, _):
        x = kx[pl.ds(i * L, L)] - dxv
        y = ky[pl.ds(i * L, L)] - dyv
        kx[pl.ds(i * L, L)] = x * cv + y * sn
        ky[pl.ds(i * L, L)] = y * cv - x * sn
        return 0

    lax.fori_loop(0, NKV, xform, 0, unroll=2)

    inf = jnp.float32(jnp.inf)
    lane = lax.iota(jnp.int32, L)
    perms = [jnp.bitwise_xor(lane, sh) for sh in (1, 2, 4, 8)]

    def xlmin(v):
        for p in perms:
            v = jnp.minimum(v, v.at[p].get(mode="promise_in_bounds"))
        return v

    def xladd(v):
        for p in perms:
            v = v + v.at[p].get(mode="promise_in_bounds")
        return v

    NQI = 2

    def insert8(s, v):
        s0, s1, s2, s3, s4, s5, s6, s7 = s
        n0 = jnp.minimum(s0, v); v = jnp.maximum(s0, v)
        n1 = jnp.minimum(s1, v); v = jnp.maximum(s1, v)
        n2 = jnp.minimum(s2, v); v = jnp.maximum(s2, v)
        n3 = jnp.minimum(s3, v); v = jnp.maximum(s3, v)
        n4 = jnp.minimum(s4, v); v = jnp.maximum(s4, v)
        n5 = jnp.minimum(s5, v); v = jnp.maximum(s5, v)
        n6 = jnp.minimum(s6, v); v = jnp.maximum(s6, v)
        n7 = jnp.minimum(s7, v)
        return (n0, n1, n2, n3, n4, n5, n6, n7)

    def merge_top8(s):
        def extract(t, carry):
            s_ = carry[:8]
            accv, remv = carry[8], carry[9]
            s0, s1, s2, s3, s4, s5, s6, s7 = s_
            me = jnp.minimum(jnp.minimum(jnp.minimum(s0, s1),
                                         jnp.minimum(s2, s3)),
                             jnp.minimum(jnp.minimum(s4, s5),
                                         jnp.minimum(s6, s7)))
            ms = xlmin(me)
            cnt = jnp.zeros((L,), jnp.float32)
            outs = []
            for r in s_:
                msk = r == ms
                cnt = cnt + jnp.where(msk, 1.0, 0.0)
                outs.append(jnp.where(msk, inf, r))
            cs = xladd(cnt)
            take = jnp.minimum(cs, remv)
            accv = accv + take * jnp.exp(ms * (-INV_SIG4))
            remv = remv - take
            return tuple(outs) + (accv, remv)

        carry0 = s + (jnp.zeros((L,), jnp.float32),
                      jnp.full((L,), float(K), jnp.float32))
        fin = lax.fori_loop(0, K, extract, carry0)
        return fin[8]

    def chunk(c, _):
        res = jnp.zeros((L,), jnp.float32)
        qxc = qx[pl.ds(c * L, L)]
        qyc = qy[pl.ds(c * L, L)]
        for j0 in range(0, L, NQI):
            qxs = []
            qys = []
            for j in range(j0, j0 + NQI):
                jv = jnp.full((L,), j, jnp.int32)
                qxs.append(qxc.at[jv].get(mode="promise_in_bounds"))
                qys.append(qyc.at[jv].get(mode="promise_in_bounds"))

            def scan_keys(i, s):
                kxv = kx[pl.ds(i * L, L)]
                kyv = ky[pl.ds(i * L, L)]
                out = ()
                for q in range(NQI):
                    dx = kxv - qxs[q]
                    dy = kyv - qys[q]
                    v = dx * dx + dy * dy
                    out = out + insert8(s[8 * q:8 * q + 8], v)
                return out

            init = (jnp.full((L,), inf),) * (8 * NQI)
            s = lax.fori_loop(0, NKV, scan_keys, init, unroll=2)

            for q in range(NQI):
                accv = merge_top8(s[8 * q:8 * q + 8])
                res = jnp.where(lane == (j0 + q), accv, res)
        sv[pl.ds(c * L, L)] = res
        return 0

    lax.fori_loop(0, QPW // L, chunk, 0)
    pltpu.sync_copy(sv, out_h.at[pl.ds(base, QPW)])


def _sc_knn(ch1x, ch1y, ch2x, ch2y, prm):
    mesh = plsc.VectorSubcoreMesh(core_axis_name="c", subcore_axis_name="s")
    f = functools.partial(
        pl.kernel,
        mesh=mesh,
        out_type=jax.ShapeDtypeStruct((QSC,), jnp.float32),
        scratch_types=[
            pltpu.VMEM((N,), jnp.float32),
            pltpu.VMEM((N,), jnp.float32),
            pltpu.VMEM((QPW,), jnp.float32),
            pltpu.VMEM((QPW,), jnp.float32),
            pltpu.VMEM((4, L), jnp.float32),
            pltpu.VMEM((QPW,), jnp.float32),
        ],
    )(_sc_body)
    return f(ch1x, ch1y, ch2x, ch2y, prm)


def _tc_body(prm_ref, qx_ref, qy_ref, kx_ref, ky_ref, o_ref):
    c = prm_ref[0, 0]
    s = prm_ref[0, 1]
    dx0 = prm_ref[0, 2]
    dy0 = prm_ref[0, 3]

    qxv = qx_ref[...]
    qyv = qy_ref[...]
    inf = jnp.float32(jnp.inf)

    def scan(j, sregs):
        xs = kx_ref[0, j] - dx0
        ys = ky_ref[0, j] - dy0
        bx = xs * c + ys * s
        by = ys * c - xs * s
        ddx = qxv - bx
        ddy = qyv - by
        v = ddx * ddx + ddy * ddy
        s0, s1, s2, s3, s4, s5, s6, s7 = sregs
        n0 = jnp.minimum(s0, v); v = jnp.maximum(s0, v)
        n1 = jnp.minimum(s1, v); v = jnp.maximum(s1, v)
        n2 = jnp.minimum(s2, v); v = jnp.maximum(s2, v)
        n3 = jnp.minimum(s3, v); v = jnp.maximum(s3, v)
        n4 = jnp.minimum(s4, v); v = jnp.maximum(s4, v)
        n5 = jnp.minimum(s5, v); v = jnp.maximum(s5, v)
        n6 = jnp.minimum(s6, v); v = jnp.maximum(s6, v)
        n7 = jnp.minimum(s7, v)
        return (n0, n1, n2, n3, n4, n5, n6, n7)

    init = (jnp.full((8, 128), inf),) * 8
    sregs = lax.fori_loop(0, N, scan, init, unroll=8)
    acc = jnp.zeros((8, 128), jnp.float32)
    for t in sregs:
        acc = acc + jnp.exp(t * (-INV_SIG4))
    o_ref[...] = acc


def _tc_knn(qx, qy, kx2d, ky2d, prm_row):
    nb = QTC // TCB
    out = pl.pallas_call(
        _tc_body,
        grid=(nb,),
        in_specs=[
            pl.BlockSpec(memory_space=pltpu.SMEM),
            pl.BlockSpec((8, 128), lambda i: (0, i)),
            pl.BlockSpec((8, 128), lambda i: (0, i)),
            pl.BlockSpec(memory_space=pltpu.SMEM),
            pl.BlockSpec(memory_space=pltpu.SMEM),
        ],
        out_specs=pl.BlockSpec((8, 128), lambda i: (0, i)),
        out_shape=jax.ShapeDtypeStruct((8, QTC // 8), jnp.float32),
    )(prm_row, qx.reshape(8, QTC // 8), qy.reshape(8, QTC // 8),
      kx2d, ky2d)
    return out.reshape(-1)


def _post_body(s_ref, o_ref):
    e = s_ref[...] * (1.0 / N)
    mask = e != 0.0
    safe = jnp.where(mask, e, 1.0)
    o_ref[0, 0] = -jnp.sum(jnp.where(mask, jnp.log(safe), 0.0))


def _post(svals):
    out = pl.pallas_call(
        _post_body,
        out_shape=jax.ShapeDtypeStruct((1, 1), jnp.float32),
        out_specs=pl.BlockSpec(memory_space=pltpu.SMEM),
    )(svals.reshape(8, N // 8))
    return out[0, 0]


def kernel(ch1, ch2, theta, d):
    ch1 = jnp.asarray(ch1, jnp.float32)
    ch2 = jnp.asarray(ch2, jnp.float32)
    th = theta[0]
    c = jnp.cos(th)
    s = jnp.sin(th)
    prm = jnp.stack([
        jnp.broadcast_to(c, (L,)),
        jnp.broadcast_to(s, (L,)),
        jnp.broadcast_to(d[0], (L,)),
        jnp.broadcast_to(d[1], (L,)),
    ])
    prm_row = jnp.stack([c, s, d[0], d[1]]).reshape(1, 4)
    ch1x = ch1[:, 0]
    ch1y = ch1[:, 1]
    ch2x = ch2[:, 0]
    ch2y = ch2[:, 1]
    s_tc = _tc_knn(ch1x[QSC:], ch1y[QSC:], ch2x.reshape(1, N),
                   ch2y.reshape(1, N), prm_row)
    s_sc = _sc_knn(ch1x[:QSC], ch1y[:QSC], ch2x, ch2y, prm)
    svals = jnp.concatenate([s_sc, s_tc])
    return _post(svals)


# speedup vs baseline: 7.7787x; 1.0903x over previous
import functools

import jax
import jax.numpy as jnp
from jax import lax
from jax.experimental import pallas as pl
from jax.experimental.pallas import tpu as pltpu
from jax.experimental.pallas import tpu_sc as plsc

N = 8192
K = 8
L = 16
NW = 32
QSC = 2048
QTC = N - QSC
QPW = QSC // NW
NKV = N // L
TCB = 1024
INV_SIG4 = 1.0 / 4.5


def _sc_body(ch1x_h, ch1y_h, ch2x_h, ch2y_h, prm_h, out_h,
             kx, ky, qx, qy, prm, sv):
    wid = lax.axis_index("s") * 2 + lax.axis_index("c")
    base = wid * QPW

    pltpu.sync_copy(ch2x_h, kx)
    pltpu.sync_copy(ch2y_h, ky)
    pltpu.sync_copy(ch1x_h.at[pl.ds(base, QPW)], qx)
    pltpu.sync_copy(ch1y_h.at[pl.ds(base, QPW)], qy)
    pltpu.sync_copy(prm_h, prm)

    cv = prm[0, :]
    sn = prm[1, :]
    dxv = prm[2, :]
    dyv = prm[3, :]

    def xform(i, _):
        x = kx[pl.ds(i * L, L)] - dxv
        y = ky[pl.ds(i * L, L)] - dyv
        kx[pl.ds(i * L, L)] = x * cv + y * sn
        ky[pl.ds(i * L, L)] = y * cv - x * sn
        return 0

    lax.fori_loop(0, NKV, xform, 0, unroll=2)

    inf = jnp.float32(jnp.inf)
    lane = lax.iota(jnp.int32, L)
    perms = [jnp.bitwise_xor(lane, sh) for sh in (1, 2, 4, 8)]

    def xlmin(v):
        for p in perms:
            v = jnp.minimum(v, v.at[p].get(mode="promise_in_bounds"))
        return v

    def xladd(v):
        for p in perms:
            v = v + v.at[p].get(mode="promise_in_bounds")
        return v

    NQI = 2

    def insert8(s, v):
        s0, s1, s2, s3, s4, s5, s6, s7 = s
        n0 = jnp.minimum(s0, v); v = jnp.maximum(s0, v)
        n1 = jnp.minimum(s1, v); v = jnp.maximum(s1, v)
        n2 = jnp.minimum(s2, v); v = jnp.maximum(s2, v)
        n3 = jnp.minimum(s3, v); v = jnp.maximum(s3, v)
        n4 = jnp.minimum(s4, v); v = jnp.maximum(s4, v)
        n5 = jnp.minimum(s5, v); v = jnp.maximum(s5, v)
        n6 = jnp.minimum(s6, v); v = jnp.maximum(s6, v)
        n7 = jnp.minimum(s7, v)
        return (n0, n1, n2, n3, n4, n5, n6, n7)

    def merge_top8(s):
        def extract(t, carry):
            s_ = carry[:8]
            accv, remv = carry[8], carry[9]
            s0, s1, s2, s3, s4, s5, s6, s7 = s_
            me = jnp.minimum(jnp.minimum(jnp.minimum(s0, s1),
                                         jnp.minimum(s2, s3)),
                             jnp.minimum(jnp.minimum(s4, s5),
                                         jnp.minimum(s6, s7)))
            ms = xlmin(me)
            cnt = jnp.zeros((L,), jnp.float32)
            outs = []
            for r in s_:
                msk = r == ms
                cnt = cnt + jnp.where(msk, 1.0, 0.0)
                outs.append(jnp.where(msk, inf, r))
            cs = xladd(cnt)
            take = jnp.minimum(cs, remv)
            accv = accv + take * jnp.exp(ms * (-INV_SIG4))
            remv = remv - take
            return tuple(outs) + (accv, remv)

        carry0 = s + (jnp.zeros((L,), jnp.float32),
                      jnp.full((L,), float(K), jnp.float32))
        fin = lax.fori_loop(0, K, extract, carry0)
        return fin[8]

    def chunk(c, _):
        res = jnp.zeros((L,), jnp.float32)
        qxc = qx[pl.ds(c * L, L)]
        qyc = qy[pl.ds(c * L, L)]
        for j0 in range(0, L, NQI):
            qxs = []
            qys = []
            for j in range(j0, j0 + NQI):
                jv = jnp.full((L,), j, jnp.int32)
                qxs.append(qxc.at[jv].get(mode="promise_in_bounds"))
                qys.append(qyc.at[jv].get(mode="promise_in_bounds"))

            def scan_keys(i, s):
                kxv = kx[pl.ds(i * L, L)]
                kyv = ky[pl.ds(i * L, L)]
                out = ()
                for q in range(NQI):
                    dx = kxv - qxs[q]
                    dy = kyv - qys[q]
                    v = dx * dx + dy * dy
                    out = out + insert8(s[8 * q:8 * q + 8], v)
                return out

            init = (jnp.full((L,), inf),) * (8 * NQI)
            s = lax.fori_loop(0, NKV, scan_keys, init, unroll=2)

            for q in range(NQI):
                accv = merge_top8(s[8 * q:8 * q + 8])
                res = jnp.where(lane == (j0 + q), accv, res)
        sv[pl.ds(c * L, L)] = res
        return 0

    lax.fori_loop(0, QPW // L, chunk, 0)
    pltpu.sync_copy(sv, out_h.at[pl.ds(base, QPW)])


def _sc_knn(ch1x, ch1y, ch2x, ch2y, prm):
    mesh = plsc.VectorSubcoreMesh(core_axis_name="c", subcore_axis_name="s")
    f = functools.partial(
        pl.kernel,
        mesh=mesh,
        out_type=jax.ShapeDtypeStruct((QSC,), jnp.float32),
        scratch_types=[
            pltpu.VMEM((N,), jnp.float32),
            pltpu.VMEM((N,), jnp.float32),
            pltpu.VMEM((QPW,), jnp.float32),
            pltpu.VMEM((QPW,), jnp.float32),
            pltpu.VMEM((4, L), jnp.float32),
            pltpu.VMEM((QPW,), jnp.float32),
        ],
    )(_sc_body)
    return f(ch1x, ch1y, ch2x, ch2y, prm)


def _tc_body(prm_ref, qx_ref, qy_ref, kx_ref, ky_ref, o_ref):
    c = prm_ref[0, 0]
    s = prm_ref[0, 1]
    dx0 = prm_ref[0, 2]
    dy0 = prm_ref[0, 3]

    qxv = qx_ref[...]
    qyv = qy_ref[...]
    inf = jnp.float32(jnp.inf)

    def scan(j, sregs):
        xs = kx_ref[0, j] - dx0
        ys = ky_ref[0, j] - dy0
        bx = xs * c + ys * s
        by = ys * c - xs * s
        ddx = qxv - bx
        ddy = qyv - by
        v = ddx * ddx + ddy * ddy
        mins = [jnp.minimum(t, v) for t in sregs]
        return (mins[0],) + tuple(
            jnp.maximum(sregs[t - 1], mins[t]) for t in range(1, 8))

    init = (jnp.full((8, 128), inf),) * 8
    sregs = lax.fori_loop(0, N, scan, init, unroll=8)
    acc = jnp.zeros((8, 128), jnp.float32)
    for t in sregs:
        acc = acc + jnp.exp(t * (-INV_SIG4))
    o_ref[...] = acc


def _tc_knn(qx, qy, kx2d, ky2d, prm_row):
    nb = QTC // TCB
    out = pl.pallas_call(
        _tc_body,
        grid=(nb,),
        in_specs=[
            pl.BlockSpec(memory_space=pltpu.SMEM),
            pl.BlockSpec((8, 128), lambda i: (0, i)),
            pl.BlockSpec((8, 128), lambda i: (0, i)),
            pl.BlockSpec(memory_space=pltpu.SMEM),
            pl.BlockSpec(memory_space=pltpu.SMEM),
        ],
        out_specs=pl.BlockSpec((8, 128), lambda i: (0, i)),
        out_shape=jax.ShapeDtypeStruct((8, QTC // 8), jnp.float32),
    )(prm_row, qx.reshape(8, QTC // 8), qy.reshape(8, QTC // 8),
      kx2d, ky2d)
    return out.reshape(-1)


def _post_body(s_ref, o_ref):
    e = s_ref[...] * (1.0 / N)
    mask = e != 0.0
    safe = jnp.where(mask, e, 1.0)
    o_ref[0, 0] = -jnp.sum(jnp.where(mask, jnp.log(safe), 0.0))


def _post(svals):
    out = pl.pallas_call(
        _post_body,
        out_shape=jax.ShapeDtypeStruct((1, 1), jnp.float32),
        out_specs=pl.BlockSpec(memory_space=pltpu.SMEM),
    )(svals.reshape(8, N // 8))
    return out[0, 0]


def kernel(ch1, ch2, theta, d):
    ch1 = jnp.asarray(ch1, jnp.float32)
    ch2 = jnp.asarray(ch2, jnp.float32)
    th = theta[0]
    c = jnp.cos(th)
    s = jnp.sin(th)
    prm = jnp.stack([
        jnp.broadcast_to(c, (L,)),
        jnp.broadcast_to(s, (L,)),
        jnp.broadcast_to(d[0], (L,)),
        jnp.broadcast_to(d[1], (L,)),
    ])
    prm_row = jnp.stack([c, s, d[0], d[1]]).reshape(1, 4)
    ch1x = ch1[:, 0]
    ch1y = ch1[:, 1]
    ch2x = ch2[:, 0]
    ch2y = ch2[:, 1]
    s_tc = _tc_knn(ch1x[QSC:], ch1y[QSC:], ch2x.reshape(1, N),
                   ch2y.reshape(1, N), prm_row)
    s_sc = _sc_knn(ch1x[:QSC], ch1y[:QSC], ch2x, ch2y, prm)
    svals = jnp.concatenate([s_sc, s_tc])
    return _post(svals)
